# baseline (speedup 1.0000x reference)
import jax
import jax.numpy as jnp
from jax.experimental import pallas as pl
from jax.experimental.pallas import tpu as pltpu

_C = 256
_HW = 180 * 180
_K = 6480
_BLK = 8192
_NB = (_HW + _BLK - 1) // _BLK


def _fused_body(bev_ref, prior_ref, out_ref, score_scr, sim_scr):
    j = pl.program_id(1)
    x = bev_ref[0]
    y = prior_ref[0]
    c = jnp.float32(_C)
    sx = jnp.sum(x, axis=0, keepdims=True)
    sy = jnp.sum(y, axis=0, keepdims=True)
    sxx = jnp.sum(x * x, axis=0, keepdims=True)
    syy = jnp.sum(y * y, axis=0, keepdims=True)
    sxy = jnp.sum(x * y, axis=0, keepdims=True)
    cxx = jnp.maximum(sxx - sx * sx / c, 0.0)
    cyy = jnp.maximum(syy - sy * sy / c, 0.0)
    cxy = sxy - sx * sy / c
    stdx = jnp.sqrt(cxx / (c - 1.0)) + 1e-6
    stdy = jnp.sqrt(cyy / (c - 1.0)) + 1e-6
    n1 = jnp.maximum(jnp.sqrt(cxx) / stdx, 1e-8)
    n2 = jnp.maximum(jnp.sqrt(cyy) / stdy, 1e-8)
    sim = (cxy / (stdx * stdy)) / (n1 * n2)
    score_scr[pl.ds(j, 1), :] = sxx
    sim_scr[pl.ds(j, 1), :] = sim

    @pl.when(j == _NB - 1)
    def _select():
        score = score_scr[...]
        simv = sim_scr[...]
        idx = (jax.lax.broadcasted_iota(jnp.int32, score.shape, 0) * _BLK
               + jax.lax.broadcasted_iota(jnp.int32, score.shape, 1))
        valid = idx < _HW
        keys = jnp.where(valid, jax.lax.bitcast_convert_type(score, jnp.int32),
                         jnp.int32(-1))

        def body(i, prefix):
            cand = prefix | jnp.left_shift(jnp.int32(1), 30 - i)
            cnt = jnp.sum((keys >= cand).astype(jnp.int32))
            return jnp.where(cnt >= _K, cand, prefix)

        v = jax.lax.fori_loop(0, 31, body, jnp.int32(0))
        n_gt = jnp.sum((keys > v).astype(jnp.int32))
        need = _K - n_gt
        eq = (keys == v) & valid
        key2 = jnp.where(eq, 32767 - idx, -1)

        def body2(i, prefix):
            cand = prefix | jnp.left_shift(jnp.int32(1), 14 - i)
            cnt = jnp.sum((key2 >= cand).astype(jnp.int32))
            return jnp.where(cnt >= need, cand, prefix)

        v2 = jax.lax.fori_loop(0, 15, body2, jnp.int32(0))
        sel = (keys > v) | (key2 >= v2)
        out_ref[...] = jnp.sum(jnp.where(sel, simv, 0.0)).reshape(1, 1, 1)


def kernel(bev_map, prior_warp, dx_m, dy_m, dtheta):
    B, C, H, W = bev_map.shape
    bev = bev_map.reshape(B, C, H * W)
    prior = prior_warp.reshape(B, C, H * W)

    sums = pl.pallas_call(
        _fused_body,
        grid=(B, _NB),
        in_specs=[
            pl.BlockSpec((1, C, _BLK), lambda b, j: (b, 0, j)),
            pl.BlockSpec((1, C, _BLK), lambda b, j: (b, 0, j)),
        ],
        out_specs=pl.BlockSpec((1, 1, 1), lambda b, j: (b, 0, 0)),
        out_shape=jax.ShapeDtypeStruct((B, 1, 1), jnp.float32),
        scratch_shapes=[
            pltpu.VMEM((_NB, _BLK), jnp.float32),
            pltpu.VMEM((_NB, _BLK), jnp.float32),
        ],
    )(bev, prior)

    align_loss = 1.0 - jnp.sum(sums) / jnp.float32(B * _K)
    reg_loss = jnp.mean(dx_m ** 2 + dy_m ** 2) + jnp.mean(dtheta ** 2)
    return align_loss + 0.1 * reg_loss


# speedup vs baseline: 1.0048x; 1.0048x over previous
import jax
import jax.numpy as jnp
from jax.experimental import pallas as pl
from jax.experimental.pallas import tpu as pltpu

_C = 256
_HW = 180 * 180
_K = 6480
_BLK = 8192
_NB = (_HW + _BLK - 1) // _BLK


def _fused_body(bev_ref, prior_ref, out_ref, score_scr, sim_scr):
    j = pl.program_id(1)
    x = bev_ref[0]
    y = prior_ref[0]
    c = jnp.float32(_C)
    sx = jnp.sum(x, axis=0, keepdims=True)
    sy = jnp.sum(y, axis=0, keepdims=True)
    sxx = jnp.sum(x * x, axis=0, keepdims=True)
    syy = jnp.sum(y * y, axis=0, keepdims=True)
    sxy = jnp.sum(x * y, axis=0, keepdims=True)
    cxx = jnp.maximum(sxx - sx * sx / c, 0.0)
    cyy = jnp.maximum(syy - sy * sy / c, 0.0)
    cxy = sxy - sx * sy / c
    stdx = jnp.sqrt(cxx / (c - 1.0)) + 1e-6
    stdy = jnp.sqrt(cyy / (c - 1.0)) + 1e-6
    n1 = jnp.maximum(jnp.sqrt(cxx) / stdx, 1e-8)
    n2 = jnp.maximum(jnp.sqrt(cyy) / stdy, 1e-8)
    sim = (cxy / (stdx * stdy)) / (n1 * n2)
    score_scr[pl.ds(j, 1), :] = sxx
    sim_scr[pl.ds(j, 1), :] = sim

    @pl.when(j == _NB - 1)
    def _select():
        score = score_scr[...]
        simv = sim_scr[...]
        idx = (jax.lax.broadcasted_iota(jnp.int32, score.shape, 0) * _BLK
               + jax.lax.broadcasted_iota(jnp.int32, score.shape, 1))
        valid = idx < _HW
        keys = jnp.where(valid, jax.lax.bitcast_convert_type(score, jnp.int32),
                         jnp.int32(-1))

        def radix_select(karr, kk, blist):
            p = jnp.int32(0)
            for b in blist:
                t = jnp.int32(0)
                for m in range(1, 8):
                    cnt = jnp.sum((karr >= (p | (m << b))).astype(jnp.int32))
                    t = t + (cnt >= kk).astype(jnp.int32)
                p = p | jnp.left_shift(t, b)
            cnt = jnp.sum((karr >= (p | 1)).astype(jnp.int32))
            return p | (cnt >= kk).astype(jnp.int32)

        v = radix_select(keys, _K, (28, 25, 22, 19, 16, 13, 10, 7, 4, 1))
        n_gt = jnp.sum((keys > v).astype(jnp.int32))
        need = _K - n_gt
        eq = (keys == v) & valid
        key2 = jnp.where(eq, 32767 - idx, -1)
        v2 = radix_select(key2, need, (12, 9, 6, 3, 0))
        sel = (keys > v) | (key2 >= v2)
        out_ref[...] = jnp.sum(jnp.where(sel, simv, 0.0)).reshape(1, 1, 1)


def kernel(bev_map, prior_warp, dx_m, dy_m, dtheta):
    B, C, H, W = bev_map.shape
    bev = bev_map.reshape(B, C, H * W)
    prior = prior_warp.reshape(B, C, H * W)

    sums = pl.pallas_call(
        _fused_body,
        grid=(B, _NB),
        in_specs=[
            pl.BlockSpec((1, C, _BLK), lambda b, j: (b, 0, j)),
            pl.BlockSpec((1, C, _BLK), lambda b, j: (b, 0, j)),
        ],
        out_specs=pl.BlockSpec((1, 1, 1), lambda b, j: (b, 0, 0)),
        out_shape=jax.ShapeDtypeStruct((B, 1, 1), jnp.float32),
        scratch_shapes=[
            pltpu.VMEM((_NB, _BLK), jnp.float32),
            pltpu.VMEM((_NB, _BLK), jnp.float32),
        ],
    )(bev, prior)

    align_loss = 1.0 - jnp.sum(sums) / jnp.float32(B * _K)
    reg_loss = jnp.mean(dx_m ** 2 + dy_m ** 2) + jnp.mean(dtheta ** 2)
    return align_loss + 0.1 * reg_loss
